# Initial kernel scaffold; baseline (speedup 1.0000x reference)
import jax
import jax.numpy as jnp
from jax.experimental import pallas as pl


def kernel(user_emb, item_emb, rows, cols):
    raise NotImplementedError("write your pallas kernel here")


# speedup vs baseline: 18.5914x; 18.5914x over previous
import functools

import jax
import jax.numpy as jnp
from jax import lax
from jax.experimental import pallas as pl
from jax.experimental.pallas import tpu as pltpu
from jax.experimental.pallas import tpu_sc as plsc

D = 32
K = 128
NTILES = 16
DW = 16

_SC_PARAMS = pltpu.CompilerParams(use_tc_tiling_on_sc=False)


def _out_copy(acc, out_hbm, c, s, half):
    big = -(-half // NTILES)
    big += (-big) % 8
    last = half - (NTILES - 1) * big

    @pl.when(s < NTILES - 1)
    def _():
        pltpu.sync_copy(
            acc.at[pl.ds(s * big, big)],
            out_hbm.at[pl.ds(c * half + s * big, big)],
        )

    @pl.when(s == NTILES - 1)
    def _():
        pltpu.sync_copy(
            acc.at[pl.ds((NTILES - 1) * big, last)],
            out_hbm.at[pl.ds(c * half + (NTILES - 1) * big, last)],
        )


def _degree_fn(n_nodes, half, nr, nb):
    zrows = nr // NTILES
    ep = nb * K * NTILES
    mesh = plsc.VectorSubcoreMesh(core_axis_name="c", subcore_axis_name="s")

    @functools.partial(
        pl.kernel,
        out_type=jax.ShapeDtypeStruct((n_nodes, DW), jnp.float32),
        mesh=mesh,
        scratch_types=[
            pltpu.VMEM((1, K), jnp.int32),
            pltpu.VMEM((K, DW), jnp.float32),
            pltpu.VMEM_SHARED((nr, DW), jnp.float32),
        ],
        compiler_params=_SC_PARAMS,
    )
    def deg_kernel(dst_hbm, ones_hbm, zeros_hbm, out_hbm, dst_v, ones_v, acc):
        c = lax.axis_index("c")
        s = lax.axis_index("s")
        pltpu.sync_copy(ones_hbm, ones_v)
        pltpu.sync_copy(zeros_hbm, acc.at[pl.ds(s * zrows, zrows)])
        plsc.subcore_barrier()
        tile_base = c * ep + s * (nb * K)

        @pl.loop(0, nb)
        def _(b):
            pltpu.sync_copy(dst_hbm.at[pl.ds(tile_base + b * K, K)], dst_v.at[0])
            pltpu.sync_copy(ones_v, acc.at[dst_v.at[0]], add=True)

        plsc.subcore_barrier()
        _out_copy(acc, out_hbm, c, s, half)

    return deg_kernel


def _propagate_fn(n_nodes, half, nr, nb):
    zrows = nr // NTILES
    ep = nb * K * NTILES
    mesh = plsc.VectorSubcoreMesh(core_axis_name="c", subcore_axis_name="s")

    @functools.partial(
        pl.kernel,
        out_type=jax.ShapeDtypeStruct((n_nodes, D), jnp.float32),
        mesh=mesh,
        scratch_types=[
            pltpu.VMEM((1, K), jnp.int32),
            pltpu.VMEM((1, K), jnp.int32),
            pltpu.VMEM((K, D), jnp.float32),
            pltpu.VMEM_SHARED((nr, D), jnp.float32),
            pltpu.SemaphoreType.DMA,
        ],
        compiler_params=_SC_PARAMS,
    )
    def prop_kernel(y_hbm, src_hbm, dst_hbm, zeros_hbm, out_hbm,
                    src_v, dst_v, rows_v, acc, sem):
        c = lax.axis_index("c")
        s = lax.axis_index("s")
        pltpu.sync_copy(zeros_hbm, acc.at[pl.ds(s * zrows, zrows)])
        plsc.subcore_barrier()
        tile_base = c * ep + s * (nb * K)

        @pl.loop(0, nb)
        def _(b):
            off = tile_base + b * K
            pltpu.sync_copy(src_hbm.at[pl.ds(off, K)], src_v.at[0])
            pltpu.sync_copy(dst_hbm.at[pl.ds(off, K)], dst_v.at[0])
            pltpu.async_copy(y_hbm.at[src_v.at[0]], rows_v, sem).wait()
            pltpu.sync_copy(rows_v, acc.at[dst_v.at[0]], add=True)

        plsc.subcore_barrier()
        _out_copy(acc, out_hbm, c, s, half)

    return prop_kernel


def _scale0(deg16, emb, br=2000):
    n = emb.shape[0]

    def body(deg_ref, emb_ref, s_ref, y_ref):
        d = deg_ref[:, 0:1]
        d = jnp.where(d == 0.0, 1e-12, d)
        sv = lax.rsqrt(d)
        s_ref[...] = sv
        y_ref[...] = emb_ref[...] * sv

    return pl.pallas_call(
        body,
        grid=(n // br,),
        in_specs=[
            pl.BlockSpec((br, DW), lambda i: (i, 0)),
            pl.BlockSpec((br, D), lambda i: (i, 0)),
        ],
        out_specs=[
            pl.BlockSpec((br, 1), lambda i: (i, 0)),
            pl.BlockSpec((br, D), lambda i: (i, 0)),
        ],
        out_shape=[
            jax.ShapeDtypeStruct((n, 1), jnp.float32),
            jax.ShapeDtypeStruct((n, D), jnp.float32),
        ],
    )(deg16, emb)


def _scale_mid(s, acc1, x0, br=2000):
    n = x0.shape[0]

    def body(s_ref, a_ref, x0_ref, y_ref, r_ref):
        sv = s_ref[...]
        x1 = sv * a_ref[...]
        y_ref[...] = sv * x1
        r_ref[...] = x0_ref[...] + x1

    return pl.pallas_call(
        body,
        grid=(n // br,),
        in_specs=[
            pl.BlockSpec((br, 1), lambda i: (i, 0)),
            pl.BlockSpec((br, D), lambda i: (i, 0)),
            pl.BlockSpec((br, D), lambda i: (i, 0)),
        ],
        out_specs=[
            pl.BlockSpec((br, D), lambda i: (i, 0)),
            pl.BlockSpec((br, D), lambda i: (i, 0)),
        ],
        out_shape=[
            jax.ShapeDtypeStruct((n, D), jnp.float32),
            jax.ShapeDtypeStruct((n, D), jnp.float32),
        ],
    )(s, acc1, x0)


def _scale_fin(s, acc2, r1, br=2000):
    n = r1.shape[0]

    def body(s_ref, a_ref, r_ref, f_ref):
        f_ref[...] = (r_ref[...] + s_ref[...] * a_ref[...]) * (1.0 / 3.0)

    return pl.pallas_call(
        body,
        grid=(n // br,),
        in_specs=[
            pl.BlockSpec((br, 1), lambda i: (i, 0)),
            pl.BlockSpec((br, D), lambda i: (i, 0)),
            pl.BlockSpec((br, D), lambda i: (i, 0)),
        ],
        out_specs=pl.BlockSpec((br, D), lambda i: (i, 0)),
        out_shape=jax.ShapeDtypeStruct((n, D), jnp.float32),
    )(s, acc2, r1)


def kernel(user_emb, item_emb, rows, cols):
    nu = user_emb.shape[0]
    ni = item_emb.shape[0]
    n = nu + ni
    e = rows.shape[0]
    half = nu

    rows32 = rows.astype(jnp.int32)
    cols32 = cols.astype(jnp.int32)

    nb = -(-e // (NTILES * K))
    ep = nb * K * NTILES
    pad = ep - e
    dummy = half
    nr = half + 8
    nr += (-nr) % (NTILES * 8)

    def _pad(x, fill):
        return jnp.concatenate([x, jnp.full((pad,), fill, jnp.int32)]) if pad else x

    src2 = jnp.concatenate([_pad(cols32 + nu, 0), _pad(rows32, 0)])
    dst2 = jnp.concatenate([_pad(rows32, dummy), _pad(cols32, dummy)])

    zrows = nr // NTILES
    zeros_d = jnp.zeros((zrows, D), jnp.float32)
    zeros_w = jnp.zeros((zrows, DW), jnp.float32)
    ones_w = jnp.ones((K, DW), jnp.float32)

    deg16 = _degree_fn(n, half, nr, nb)(dst2, ones_w, zeros_w)

    x0 = jnp.concatenate([user_emb, item_emb], axis=0)
    s, y0 = _scale0(deg16, x0)

    prop = _propagate_fn(n, half, nr, nb)
    acc1 = prop(y0, src2, dst2, zeros_d)
    y1, r1 = _scale_mid(s, acc1, x0)
    acc2 = prop(y1, src2, dst2, zeros_d)
    final = _scale_fin(s, acc2, r1)

    return final[:nu], final[nu:]
